# baseline (speedup 1.0000x reference)
import functools

import jax
import jax.numpy as jnp
from jax import lax
from jax.experimental import pallas as pl
from jax.experimental.pallas import tpu as pltpu
from jax.experimental.pallas import tpu_sc as plsc

_NC, _NS, _L = 2, 16, 16
_NW = _NC * _NS


@functools.partial(jax.jit, static_argnums=(4, 5, 6, 7))
def _embed_add(ids, pids, wtab, ptab_bf, N, V, P, H):
    rpw = N // _NW
    C = 16
    n_chunks = rpw // C
    mesh = plsc.VectorSubcoreMesh(
        core_axis_name="c", subcore_axis_name="s",
        num_cores=_NC, num_subcores=_NS)

    @functools.partial(
        pl.kernel,
        out_type=jax.ShapeDtypeStruct((N, H), jnp.float32),
        mesh=mesh,
        scratch_types=[
            pltpu.VMEM((rpw,), jnp.int32),
            pltpu.VMEM((rpw,), jnp.int32),
            [pltpu.VMEM((C, H), jnp.float32)] * 2,
            [pltpu.VMEM((C, H // 2), jnp.int32)] * 2,
            [pltpu.VMEM((C, H), jnp.float32)] * 2,
            [pltpu.SemaphoreType.DMA] * 2,
            [pltpu.SemaphoreType.DMA] * 2,
        ],
        compiler_params=pltpu.CompilerParams(needs_layout_passes=False),
    )
    def k(ids_hbm, pids_hbm, wtab_hbm, ptab_hbm, out_hbm,
          idx_v, pidx_v, bw, bp, bo, gsem, wsem):
        wid = lax.axis_index("s") * _NC + lax.axis_index("c")
        base = wid * rpw
        pltpu.sync_copy(ids_hbm.at[pl.ds(base, rpw)], idx_v)
        pltpu.sync_copy(pids_hbm.at[pl.ds(base, rpw)], pidx_v)

        def issue_gathers(c, b):
            pltpu.async_copy(
                wtab_hbm.at[idx_v.at[pl.ds(c * C, C)]], bw[b], gsem[b])
            pltpu.async_copy(
                ptab_hbm.at[pidx_v.at[pl.ds(c * C, C)]], bp[b], gsem[b])

        def drain_gathers(b):
            pltpu.make_async_copy(wtab_hbm.at[pl.ds(0, C)], bw[b],
                                  gsem[b]).wait()
            pltpu.make_async_copy(ptab_hbm.at[pl.ds(0, C)], bp[b],
                                  gsem[b]).wait()

        issue_gathers(0, 0)
        issue_gathers(1, 1)

        @pl.loop(0, n_chunks, step=2)
        def _(c0):
            for b in range(2):
                c = c0 + b
                drain_gathers(b)

                @pl.when(c >= 2)
                def _():
                    pltpu.make_async_copy(
                        bo[b], out_hbm.at[pl.ds(0, C)], wsem[b]).wait()

                @plsc.parallel_loop(0, C)
                def _(r):
                    for g in range(H // (2 * _L)):
                        v = plsc.bitcast(bp[b][r, pl.ds(g * _L, _L)],
                                         jnp.bfloat16)
                        lo, hi = plsc.unpack(
                            v, format=plsc.PackFormat.INTERLEAVED)
                        sl_lo = pl.ds(g * _L, _L)
                        sl_hi = pl.ds(H // 2 + g * _L, _L)
                        bo[b][r, sl_lo] = bw[b][r, sl_lo] + lo
                        bo[b][r, sl_hi] = bw[b][r, sl_hi] + hi

                @pl.when(c + 2 < n_chunks)
                def _():
                    issue_gathers(c + 2, b)

                pltpu.async_copy(
                    bo[b], out_hbm.at[pl.ds(base + c * C, C)], wsem[b])

        for b in range(2):
            pltpu.make_async_copy(bo[b], out_hbm.at[pl.ds(0, C)],
                                  wsem[b]).wait()

    return k(ids, pids, wtab, ptab_bf)


@functools.partial(jax.jit, static_argnums=(1, 2))
def _pack_pos(ptab, P, H):
    Hh = H // 2
    RB = 256

    def body(x_ref, o_ref):
        x = jax.lax.bitcast_convert_type(x_ref[...], jnp.int32)

        def rne16(y):
            return (y + 0x7FFF + ((y >> 16) & 1)) >> 16

        lo = rne16(x[:, :Hh]) & 0xFFFF
        hi = rne16(x[:, Hh:]) << 16
        o_ref[...] = lo | hi

    return pl.pallas_call(
        body,
        grid=(P // RB,),
        in_specs=[pl.BlockSpec((RB, H), lambda i: (i, 0))],
        out_specs=pl.BlockSpec((RB, Hh), lambda i: (i, 0)),
        out_shape=jax.ShapeDtypeStruct((P, Hh), jnp.int32),
    )(ptab)


def kernel(input_ids, position_ids, word_embeddings, position_embeddings):
    B, S = input_ids.shape
    V, H = word_embeddings.shape
    P = position_embeddings.shape[0]
    N = B * S
    ids = input_ids.reshape(N).astype(jnp.int32)
    pids = position_ids.reshape(N).astype(jnp.int32)
    ptab_i = _pack_pos(position_embeddings, P, H)
    out = _embed_add(ids, pids, word_embeddings, ptab_i, N, V, P, H)
    return out.reshape(B, S, H)


# speedup vs baseline: 1.0648x; 1.0648x over previous
import functools

import jax
import jax.numpy as jnp
from jax import lax
from jax.experimental import pallas as pl
from jax.experimental.pallas import tpu as pltpu
from jax.experimental.pallas import tpu_sc as plsc

_NC, _NS, _L = 2, 16, 16
_NW = _NC * _NS


@functools.partial(jax.jit, static_argnums=(4, 5, 6, 7))
def _embed_add(ids, pids, wtab, ptab, N, V, P, H):
    rpw = N // _NW
    C = 16
    n_chunks = rpw // C
    mesh = plsc.VectorSubcoreMesh(
        core_axis_name="c", subcore_axis_name="s",
        num_cores=_NC, num_subcores=_NS)
    row_t = jax.ShapeDtypeStruct((C, H), jnp.float32)

    @functools.partial(
        pl.kernel,
        out_type=jax.ShapeDtypeStruct((N, H), jnp.float32),
        mesh=mesh,
        scratch_types=[
            pltpu.VMEM((rpw,), jnp.int32),
            pltpu.VMEM((rpw,), jnp.int32),
            [pltpu.VMEM((C, H), jnp.float32)] * 2,
            [pltpu.VMEM((C, H), jnp.float32)] * 2,
            [pltpu.VMEM((C, H), jnp.float32)] * 2,
            [pltpu.SemaphoreType.DMA] * 2,
            [pltpu.SemaphoreType.DMA] * 2,
        ],
    )
    def k(ids_hbm, pids_hbm, wtab_hbm, ptab_hbm, out_hbm,
          idx_v, pidx_v, bw, bp, bo, gsem, wsem):
        wid = lax.axis_index("s") * _NC + lax.axis_index("c")
        base = wid * rpw
        pltpu.sync_copy(ids_hbm.at[pl.ds(base, rpw)], idx_v)
        pltpu.sync_copy(pids_hbm.at[pl.ds(base, rpw)], pidx_v)

        def issue_gathers(c, b):
            pltpu.async_copy(
                wtab_hbm.at[idx_v.at[pl.ds(c * C, C)]], bw[b], gsem[b])
            pltpu.async_copy(
                ptab_hbm.at[pidx_v.at[pl.ds(c * C, C)]], bp[b], gsem[b])

        def drain_gathers(b):
            pltpu.make_async_copy(wtab_hbm.at[pl.ds(0, C)], bw[b],
                                  gsem[b]).wait()
            pltpu.make_async_copy(ptab_hbm.at[pl.ds(0, C)], bp[b],
                                  gsem[b]).wait()

        issue_gathers(0, 0)
        issue_gathers(1, 1)

        @pl.loop(0, n_chunks, step=2)
        def _(c0):
            for b in range(2):
                c = c0 + b
                drain_gathers(b)

                @pl.when(c >= 2)
                def _():
                    pltpu.make_async_copy(
                        bo[b], out_hbm.at[pl.ds(0, C)], wsem[b]).wait()

                @plsc.parallel_loop(0, C * H // (8 * _L))
                def _(i):
                    r = i >> 3
                    g0 = (i & 7) * 8 * _L
                    for u in range(8):
                        sl = pl.ds(g0 + u * _L, _L)
                        bo[b][r, sl] = bw[b][r, sl] + bp[b][r, sl]

                @pl.when(c + 2 < n_chunks)
                def _():
                    issue_gathers(c + 2, b)

                pltpu.async_copy(
                    bo[b], out_hbm.at[pl.ds(base + c * C, C)], wsem[b])

        for b in range(2):
            pltpu.make_async_copy(bo[b], out_hbm.at[pl.ds(0, C)],
                                  wsem[b]).wait()

    return k(ids, pids, wtab, ptab)


def kernel(input_ids, position_ids, word_embeddings, position_embeddings):
    B, S = input_ids.shape
    V, H = word_embeddings.shape
    P = position_embeddings.shape[0]
    N = B * S
    ids = input_ids.reshape(N).astype(jnp.int32)
    pids = position_ids.reshape(N).astype(jnp.int32)
    out = _embed_add(ids, pids, word_embeddings, position_embeddings,
                     N, V, P, H)
    return out.reshape(B, S, H)
